# Initial kernel scaffold; baseline (speedup 1.0000x reference)
import jax
import jax.numpy as jnp
from jax.experimental import pallas as pl


def kernel(x, Wr, W1, b1, W2, b2):
    raise NotImplementedError("write your pallas kernel here")


# speedup vs baseline: 3.2589x; 3.2589x over previous
import functools

import jax
import jax.numpy as jnp
from jax.experimental import pallas as pl
from jax.experimental.pallas import tpu as pltpu

_T_R = 512
_T_B = 256


def _router_kernel(x_ref, wr_ref, probs_ref, idx_ref, w_ref, *, num_experts):
    xb = x_ref[...]
    logits = jnp.dot(xb, wr_ref[...], preferred_element_type=jnp.float32)
    m = jnp.max(logits, axis=-1, keepdims=True)
    p = jnp.exp(logits - m)
    probs = p / jnp.sum(p, axis=-1, keepdims=True)
    probs_ref[...] = probs
    iota = jax.lax.broadcasted_iota(jnp.int32, probs.shape, 1)
    m1 = jnp.max(probs, axis=-1, keepdims=True)
    i1 = jnp.min(jnp.where(probs == m1, iota, num_experts), axis=-1,
                 keepdims=True)
    masked = jnp.where(iota == i1, -1.0, probs)
    m2 = jnp.max(masked, axis=-1, keepdims=True)
    i2 = jnp.min(jnp.where(masked == m2, iota, num_experts), axis=-1,
                 keepdims=True)
    s = m1 + m2
    idx_ref[...] = jnp.concatenate([i1, i2], axis=-1)
    w_ref[...] = jnp.concatenate([m1 / s, m2 / s], axis=-1)


def _ffn_kernel(be_ref, x_ref, w1_ref, b1_ref, w2_ref, b2_ref, wg_ref, o_ref):
    g = pl.program_id(0)

    @pl.when(be_ref[g] >= 0)
    def _():
        xb = x_ref[...]
        h = jnp.dot(xb, w1_ref[0], preferred_element_type=jnp.float32)
        h = h + b1_ref[0]
        a = 0.5 * h * (1.0 + jax.lax.erf(h * 0.7071067811865476))
        y = jnp.dot(a, w2_ref[0], preferred_element_type=jnp.float32)
        y = y + b2_ref[0]
        o_ref[...] = y * wg_ref[...]


def kernel(x, Wr, W1, b1, W2, b2):
    B, S, D = x.shape
    E = Wr.shape[1]
    H = W1.shape[2]
    K = 2
    N = B * S
    P = N * K
    P_MAX = P + E * _T_B
    G_MAX = P_MAX // _T_B

    x2d = x.reshape(N, D)

    probs, sel, w = pl.pallas_call(
        functools.partial(_router_kernel, num_experts=E),
        grid=(N // _T_R,),
        in_specs=[
            pl.BlockSpec((_T_R, D), lambda i: (i, 0)),
            pl.BlockSpec((D, E), lambda i: (0, 0)),
        ],
        out_specs=[
            pl.BlockSpec((_T_R, E), lambda i: (i, 0)),
            pl.BlockSpec((_T_R, K), lambda i: (i, 0)),
            pl.BlockSpec((_T_R, K), lambda i: (i, 0)),
        ],
        out_shape=[
            jax.ShapeDtypeStruct((N, E), jnp.float32),
            jax.ShapeDtypeStruct((N, K), jnp.int32),
            jax.ShapeDtypeStruct((N, K), jnp.float32),
        ],
    )(x2d, Wr)

    e_flat = sel.reshape(P)
    onehot = (e_flat[:, None] == jnp.arange(E, dtype=jnp.int32)[None, :]
              ).astype(jnp.int32)
    ranks_all = jnp.cumsum(onehot, axis=0) - onehot
    rank = jnp.take_along_axis(ranks_all, e_flat[:, None], axis=1)[:, 0]
    counts = jnp.sum(onehot, axis=0)
    padded = ((counts + _T_B - 1) // _T_B) * _T_B
    ends = jnp.cumsum(padded)
    offs = ends - padded
    pos = offs[e_flat] + rank

    blk_start = jnp.arange(G_MAX, dtype=jnp.int32) * _T_B
    be = jnp.sum((blk_start[:, None] >= ends[None, :]).astype(jnp.int32),
                 axis=1)
    block_expert = jnp.where(be < E, be, -1).astype(jnp.int32)

    row_token = jnp.zeros((P_MAX,), jnp.int32).at[pos].set(
        jnp.arange(P, dtype=jnp.int32) // K)
    wg = jnp.zeros((P_MAX,), jnp.float32).at[pos].set(w.reshape(P))

    xg = x2d[row_token]

    grid_spec = pltpu.PrefetchScalarGridSpec(
        num_scalar_prefetch=1,
        grid=(G_MAX,),
        in_specs=[
            pl.BlockSpec((_T_B, D), lambda g, be: (g, 0)),
            pl.BlockSpec((1, D, H),
                         lambda g, be: (jnp.maximum(be[g], 0), 0, 0)),
            pl.BlockSpec((1, 1, H),
                         lambda g, be: (jnp.maximum(be[g], 0), 0, 0)),
            pl.BlockSpec((1, H, D),
                         lambda g, be: (jnp.maximum(be[g], 0), 0, 0)),
            pl.BlockSpec((1, 1, D),
                         lambda g, be: (jnp.maximum(be[g], 0), 0, 0)),
            pl.BlockSpec((_T_B, 1), lambda g, be: (g, 0)),
        ],
        out_specs=pl.BlockSpec((_T_B, D), lambda g, be: (g, 0)),
    )
    yg = pl.pallas_call(
        _ffn_kernel,
        grid_spec=grid_spec,
        out_shape=jax.ShapeDtypeStruct((P_MAX, D), jnp.float32),
    )(block_expert, xg, W1, b1.reshape(E, 1, H), W2, b2.reshape(E, 1, D),
      wg[:, None])

    pos2 = pos.reshape(N, K)
    out2d = yg[pos2[:, 0]] + yg[pos2[:, 1]]

    return (out2d.reshape(B, S, D), probs.reshape(B, S, E),
            sel.reshape(B, S, K), w.reshape(B, S, K))
